# baseline (speedup 1.0000x reference)
import functools

import jax
import jax.numpy as jnp
from jax import lax
from jax.experimental import pallas as pl
from jax.experimental.pallas import tpu as pltpu
from jax.experimental.pallas import tpu_sc as plsc

N_ROWS = 4096
N_CLASSES = 401
MAX_ATTRS = 16
BLOCK_ROWS = 512
N_BLOCKS = N_ROWS // BLOCK_ROWS

SC_CORES = 2
SC_SUBCORES = 16
NW = SC_CORES * SC_SUBCORES
RPW = N_ROWS // NW
GROUPS = RPW // 16


def _sc_body(logits, attrs_flat, aux, slab, attr_buf, out_buf):
    wid = lax.axis_index("s") * SC_CORES + lax.axis_index("c")
    rbase = pl.multiple_of(wid * RPW, RPW)
    iota = lax.iota(jnp.int32, 16)

    pltpu.sync_copy(attrs_flat.at[pl.ds(rbase * MAX_ATTRS, RPW * MAX_ATTRS)],
                    attr_buf)
    pltpu.sync_copy(logits.at[pl.ds(rbase, RPW), :], slab)

    def col(g, j):
        return plsc.load_gather(
            attr_buf, [(g * 16 + iota) * MAX_ATTRS + j])

    ones = jnp.ones((16,), jnp.float32)
    zeros = jnp.zeros((16,), jnp.float32)

    def group_body(g, _):
        rows = g * 16 + iota
        ids_list = [col(g, j) for j in range(MAX_ATTRS)]
        gacc = zeros
        dacc = zeros
        cacc = zeros
        for j in range(MAX_ATTRS):
            ids_j = ids_list[j]
            v_j = plsc.load_gather(slab, [rows, ids_j])
            nz = ids_j != 0
            first = nz
            for k in range(j):
                first = first & (ids_j != ids_list[k])
            fj = jnp.where(first, ones, zeros)
            gacc = gacc + fj * v_j
            dacc = dacc + fj
            cacc = cacc + jnp.where(nz, ones, zeros)
        rcp = 1.0 / jnp.maximum(cacc, ones)
        out_buf[pl.ds(g * 16, 16)] = dacc * rcp
        out_buf[pl.ds(RPW + g * 16, 16)] = gacc * rcp
        return _

    lax.fori_loop(0, GROUPS, group_body, None)

    b = wid // (BLOCK_ROWS // RPW)
    lo = pl.multiple_of((wid % (BLOCK_ROWS // RPW)) * RPW, RPW)
    pltpu.sync_copy(out_buf.at[pl.ds(0, RPW)], aux.at[b, 0, pl.ds(lo, RPW)])
    pltpu.sync_copy(out_buf.at[pl.ds(RPW, RPW)],
                    aux.at[b, 0, pl.ds(BLOCK_ROWS + lo, RPW)])


@functools.partial(
    pl.kernel,
    out_type=jax.ShapeDtypeStruct((N_BLOCKS, 1, BLOCK_ROWS * 2), jnp.float32),
    mesh=plsc.VectorSubcoreMesh(core_axis_name="c", subcore_axis_name="s"),
    scratch_types=[
        pltpu.VMEM((RPW, N_CLASSES), jnp.float32),
        pltpu.VMEM((RPW * MAX_ATTRS,), jnp.int32),
        pltpu.VMEM((RPW * 2,), jnp.float32),
    ],
    compiler_params=pltpu.CompilerParams(needs_layout_passes=False),
)
def _sc_gather(logits, attrs_flat, aux, slab, attr_buf, out_buf):
    _sc_body(logits, attrs_flat, aux, slab, attr_buf, out_buf)


def _tc_body(logits_ref, aux_ref, out_ref):
    @pl.when(pl.program_id(0) == 0)
    def _():
        out_ref[...] = jnp.zeros((1, 1), jnp.float32)

    x = logits_ref[...]
    mx = jnp.max(x, axis=1, keepdims=True)
    se = jnp.sum(jnp.exp(x - mx), axis=1, keepdims=True)
    lse = mx + jnp.log(se)
    dn = aux_ref[0, 0:1, 0:BLOCK_ROWS]
    gn = aux_ref[0, 0:1, BLOCK_ROWS:2 * BLOCK_ROWS]
    p1 = jax.lax.dot_general(dn, lse, (((1,), (0,)), ((), ())),
                             preferred_element_type=jnp.float32)
    p2 = jnp.sum(gn, keepdims=True).reshape(1, 1)
    out_ref[...] += (p1 - p2) * (1.0 / N_ROWS)


def kernel(attribute_logits, attributes):
    aux = _sc_gather(attribute_logits, attributes.reshape(-1))
    out = pl.pallas_call(
        _tc_body,
        grid=(N_BLOCKS,),
        in_specs=[
            pl.BlockSpec((BLOCK_ROWS, N_CLASSES), lambda i: (i, 0)),
            pl.BlockSpec((1, 1, 2 * BLOCK_ROWS), lambda i: (i, 0, 0)),
        ],
        out_specs=pl.BlockSpec((1, 1), lambda i: (0, 0)),
        out_shape=jax.ShapeDtypeStruct((1, 1), jnp.float32),
    )(attribute_logits, aux)
    return out[0, 0]


# speedup vs baseline: 1.0156x; 1.0156x over previous
import functools

import jax
import jax.numpy as jnp
from jax import lax
from jax.experimental import pallas as pl
from jax.experimental.pallas import tpu as pltpu
from jax.experimental.pallas import tpu_sc as plsc

N_ROWS = 4096
N_CLASSES = 401
MAX_ATTRS = 16
BLOCK_ROWS = 512
N_BLOCKS = N_ROWS // BLOCK_ROWS

SC_CORES = 2
SC_SUBCORES = 16
NW = SC_CORES * SC_SUBCORES
RPW = N_ROWS // NW
GROUPS = RPW // 16


def _slab_copy(logits, slab, sem, rbase, g):
    return pltpu.make_async_copy(
        logits.at[pl.ds(rbase + g * 16, 16), :],
        slab.at[pl.ds(g * 16, 16), :], sem.at[g])


def _sc_body(logits, attrs_flat, aux, slab, attr_buf, out_buf, sem):
    wid = lax.axis_index("s") * SC_CORES + lax.axis_index("c")
    rbase = pl.multiple_of(wid * RPW, RPW)
    iota = lax.iota(jnp.int32, 16)

    for g in range(GROUPS):
        _slab_copy(logits, slab, sem, rbase, g).start()
    pltpu.sync_copy(attrs_flat.at[pl.ds(rbase * MAX_ATTRS, RPW * MAX_ATTRS)],
                    attr_buf)

    def col(g, j):
        return plsc.load_gather(
            attr_buf, [(g * 16 + iota) * MAX_ATTRS + j])

    ones = jnp.ones((16,), jnp.float32)
    zeros = jnp.zeros((16,), jnp.float32)

    def group_body(g, _):
        _slab_copy(logits, slab, sem, rbase, g).wait()
        rows = g * 16 + iota
        ids_list = [col(g, j) for j in range(MAX_ATTRS)]
        gacc = zeros
        dacc = zeros
        cacc = zeros
        for j in range(MAX_ATTRS):
            ids_j = ids_list[j]
            v_j = plsc.load_gather(slab, [rows, ids_j])
            nz = ids_j != 0
            first = nz
            for k in range(j):
                first = first & (ids_j != ids_list[k])
            fj = jnp.where(first, ones, zeros)
            gacc = gacc + fj * v_j
            dacc = dacc + fj
            cacc = cacc + jnp.where(nz, ones, zeros)
        rcp = 1.0 / jnp.maximum(cacc, ones)
        out_buf[pl.ds(g * 16, 16)] = dacc * rcp
        out_buf[pl.ds(RPW + g * 16, 16)] = gacc * rcp
        return _

    lax.fori_loop(0, GROUPS, group_body, None)

    b = wid // (BLOCK_ROWS // RPW)
    lo = pl.multiple_of((wid % (BLOCK_ROWS // RPW)) * RPW, RPW)
    pltpu.sync_copy(out_buf.at[pl.ds(0, RPW)], aux.at[b, 0, pl.ds(lo, RPW)])
    pltpu.sync_copy(out_buf.at[pl.ds(RPW, RPW)],
                    aux.at[b, 0, pl.ds(BLOCK_ROWS + lo, RPW)])


@functools.partial(
    pl.kernel,
    out_type=jax.ShapeDtypeStruct((N_BLOCKS, 1, BLOCK_ROWS * 2), jnp.float32),
    mesh=plsc.VectorSubcoreMesh(core_axis_name="c", subcore_axis_name="s"),
    scratch_types=[
        pltpu.VMEM((RPW, N_CLASSES), jnp.float32),
        pltpu.VMEM((RPW * MAX_ATTRS,), jnp.int32),
        pltpu.VMEM((RPW * 2,), jnp.float32),
        pltpu.SemaphoreType.DMA((GROUPS,)),
    ],
    compiler_params=pltpu.CompilerParams(needs_layout_passes=False),
)
def _sc_gather(logits, attrs_flat, aux, slab, attr_buf, out_buf, sem):
    _sc_body(logits, attrs_flat, aux, slab, attr_buf, out_buf, sem)


def _tc_body(logits_ref, aux_ref, out_ref):
    @pl.when(pl.program_id(0) == 0)
    def _():
        out_ref[...] = jnp.zeros((1, 1), jnp.float32)

    x = logits_ref[...]
    mx = jnp.max(x, axis=1, keepdims=True)
    se = jnp.sum(jnp.exp(x - mx), axis=1, keepdims=True)
    lse = mx + jnp.log(se)
    dn = aux_ref[0, 0:1, 0:BLOCK_ROWS]
    gn = aux_ref[0, 0:1, BLOCK_ROWS:2 * BLOCK_ROWS]
    p1 = jax.lax.dot_general(dn, lse, (((1,), (0,)), ((), ())),
                             preferred_element_type=jnp.float32)
    p2 = jnp.sum(gn, keepdims=True).reshape(1, 1)
    out_ref[...] += (p1 - p2) * (1.0 / N_ROWS)


def kernel(attribute_logits, attributes):
    aux = _sc_gather(attribute_logits, attributes.reshape(-1))
    out = pl.pallas_call(
        _tc_body,
        grid=(N_BLOCKS,),
        in_specs=[
            pl.BlockSpec((BLOCK_ROWS, N_CLASSES), lambda i: (i, 0)),
            pl.BlockSpec((1, 1, 2 * BLOCK_ROWS), lambda i: (i, 0, 0)),
        ],
        out_specs=pl.BlockSpec((1, 1), lambda i: (0, 0)),
        out_shape=jax.ShapeDtypeStruct((1, 1), jnp.float32),
    )(attribute_logits, aux)
    return out[0, 0]
